# Initial kernel scaffold; baseline (speedup 1.0000x reference)
import jax
import jax.numpy as jnp
from jax.experimental import pallas as pl


def kernel(inputs, rates):
    raise NotImplementedError("write your pallas kernel here")


# speedup vs baseline: 2.6370x; 2.6370x over previous
import functools

import jax
import jax.numpy as jnp
from jax.experimental import pallas as pl
from jax.experimental.pallas import tpu as pltpu

B = 256
N = 65536
K = 10
BN = 8192
G = N // BN


def _body(x_ref, rt_ref, o_ref, occ_ref):
    i = pl.program_id(0)

    @pl.when(i == 0)
    def _init():
        o_ref[...] = jnp.zeros_like(o_ref)
        occ_ref[...] = jnp.zeros_like(occ_ref)

    r = rt_ref[...]
    denom = jnp.maximum(jnp.sum(jnp.abs(r), axis=0, keepdims=True), 1e-12)
    p = r / denom
    m = jnp.max(p, axis=0, keepdims=True)
    row = jax.lax.broadcasted_iota(jnp.int32, p.shape, 0)
    ismax = p == m
    amax = jnp.min(jnp.where(ismax, row, K), axis=0, keepdims=True)
    onehot = row == amax
    assoc = jnp.where(onehot, p, 0.0)

    x = x_ref[...].astype(jnp.bfloat16)
    part = jax.lax.dot_general(
        x, assoc.astype(jnp.bfloat16),
        dimension_numbers=(((1,), (1,)), ((), ())),
        preferred_element_type=jnp.float32,
    )
    o_ref[...] += part
    occ_ref[0:1, :] += jnp.sum(onehot.astype(jnp.float32), axis=1)[None, :]

    @pl.when(i == G - 1)
    def _finish():
        occ = occ_ref[0:1, :]
        y = o_ref[...]
        o_ref[...] = jnp.where(occ > 0.0, y / occ, 0.0)


@jax.jit
def kernel(inputs, rates):
    rates_t = rates.T
    out = pl.pallas_call(
        _body,
        grid=(G,),
        in_specs=[
            pl.BlockSpec((B, BN), lambda i: (0, i)),
            pl.BlockSpec((K, BN), lambda i: (0, i)),
        ],
        out_specs=pl.BlockSpec((B, K), lambda i: (0, 0)),
        out_shape=jax.ShapeDtypeStruct((B, K), jnp.float32),
        scratch_shapes=[pltpu.VMEM((1, K), jnp.float32)],
        compiler_params=pltpu.CompilerParams(
            dimension_semantics=("arbitrary",),
        ),
    )(inputs, rates_t)
    return out
